# baseline (device time: 8063 ns/iter reference)
import jax
import jax.numpy as jnp
from jax import lax
from jax.experimental import pallas as pl
from jax.experimental.pallas import tpu as pltpu

N_DEV = 4


def _ce(x, idx, j, k, flip=None):
    return _ce_asc(x, idx, j, (idx & k) == 0, flip)


def _ce_asc(x, idx, j, asc, flip=None):
    down = jnp.roll(x, -j, axis=0)
    up = jnp.roll(x, j, axis=0)
    lower = (idx & j) == 0
    partner = jnp.where(lower, down, up)
    take_min = asc == lower
    if flip is not None:
        take_min = jnp.logical_xor(take_min, flip)
    return jnp.where(take_min, jnp.minimum(x, partner), jnp.maximum(x, partner))


def kernel(x):
    m_per, n = x.shape
    m_full = N_DEV * m_per

    def body(x_ref, out_ref, chunk_ref, full_ref, send_sems, recv_sems):
        my_pos = lax.axis_index("i")

        barrier_sem = pltpu.get_barrier_semaphore()
        for d in (2, 1, 3):
            pl.semaphore_signal(
                barrier_sem, inc=1,
                device_id=((my_pos + d) % N_DEV,),
                device_id_type=pl.DeviceIdType.MESH,
            )

        xv = x_ref[:, :].astype(jnp.bfloat16)
        m_h = m_per // 2
        flip = (my_pos % 2) == 1
        xvf = jnp.concatenate([xv[:m_h, :], xv[m_h:, :]], axis=1)
        idx_l = lax.broadcasted_iota(jnp.int32, (m_h, 2 * n), 0)
        asc_ll = lax.broadcasted_iota(jnp.int32, (m_h, 2 * n), 1) < n
        k = 2
        while k <= m_h:
            j = k // 2
            while j >= 1:
                asc = asc_ll if k == m_h else (idx_l & k) == 0
                xvf = _ce_asc(xvf, idx_l, j, asc, flip)
                j //= 2
            k *= 2
        a = xvf[:, :n]
        b = xvf[:, n:]
        blo = jnp.where(flip, jnp.maximum(a, b), jnp.minimum(a, b))
        bhi = jnp.where(flip, jnp.minimum(a, b), jnp.maximum(a, b))
        xvf = jnp.concatenate([blo, bhi], axis=1)
        j = m_h // 2
        while j >= 1:
            xvf = _ce_asc(xvf, idx_l, j, True, flip)
            j //= 2
        xv = jnp.concatenate([xvf[:, :n], xvf[:, n:]], axis=0)
        chunk_ref[:, :] = xv
        full_ref[pl.ds(my_pos * m_per, m_per), :] = xv

        pl.semaphore_wait(barrier_sem, N_DEV - 1)

        sends = []
        for d in (2, 1, 3):
            rdma = pltpu.make_async_remote_copy(
                src_ref=chunk_ref,
                dst_ref=full_ref.at[pl.ds(my_pos * m_per, m_per)],
                send_sem=send_sems.at[d - 1],
                recv_sem=recv_sems.at[d - 1],
                device_id=((my_pos + d) % N_DEV,),
                device_id_type=pl.DeviceIdType.MESH,
            )
            rdma.start()
            sends.append(rdma)

        def wait_chunk(d):
            origin = (my_pos - d) % N_DEV
            recv = pltpu.make_async_remote_copy(
                src_ref=chunk_ref,
                dst_ref=full_ref.at[pl.ds(origin * m_per, m_per)],
                send_sem=send_sems.at[d - 1],
                recv_sem=recv_sems.at[d - 1],
                device_id=(my_pos,),
                device_id_type=pl.DeviceIdType.MESH,
            )
            recv.wait_recv()

        wait_chunk(1)
        wait_chunk(3)
        m_half = 2 * m_per
        half = my_pos // 2
        idx_h = lax.broadcasted_iota(jnp.int32, (m_half, n), 0)
        xm = full_ref[pl.ds(half * m_half, m_half), :]
        flip_mine = half == 1
        j = 128
        while j >= 1:
            xm = _ce_asc(xm, idx_h, j, True, flip_mine)
            j //= 2

        wait_chunk(2)
        xo = full_ref[pl.ds((1 - half) * m_half, m_half), :]
        flip_other = half == 0
        j = 128
        while j >= 1:
            xo = _ce_asc(xo, idx_h, j, True, flip_other)
            j //= 2

        is_lo_half = my_pos < 2
        xh = jnp.where(is_lo_half, jnp.minimum(xm, xo), jnp.maximum(xm, xo))

        a = xh[:m_per, :]
        b = xh[m_per:, :]
        qlo = jnp.minimum(a, b)
        qhi = jnp.maximum(a, b)
        xqf = jnp.concatenate([qlo, qhi], axis=1)

        idx_q = lax.broadcasted_iota(jnp.int32, (m_per, 2 * n), 0)
        j = 64
        while j >= 1:
            xqf = _ce_asc(xqf, idx_q, j, True)
            j //= 2

        is_lo_q = (my_pos % 2) == 0
        out_ref[:, :] = jnp.where(is_lo_q, xqf[:, :n], xqf[:, n:])

        for rdma in sends:
            rdma.wait_send()

    return pl.pallas_call(
        body,
        out_shape=jax.ShapeDtypeStruct((m_per, n), jnp.bfloat16),
        in_specs=[pl.BlockSpec(memory_space=pltpu.VMEM)],
        out_specs=pl.BlockSpec(memory_space=pltpu.VMEM),
        scratch_shapes=[
            pltpu.VMEM((m_per, n), jnp.bfloat16),
            pltpu.VMEM((m_full, n), jnp.bfloat16),
            pltpu.SemaphoreType.DMA((N_DEV - 1,)),
            pltpu.SemaphoreType.DMA((N_DEV - 1,)),
        ],
        compiler_params=pltpu.CompilerParams(collective_id=0),
    )(x)


# device time: 7764 ns/iter; 1.0385x vs baseline; 1.0385x over previous
import jax
import jax.numpy as jnp
from jax import lax
from jax.experimental import pallas as pl
from jax.experimental.pallas import tpu as pltpu

N_DEV = 4


def _ce(x, idx, j, k, flip=None):
    return _ce_asc(x, idx, j, (idx & k) == 0, flip)


def _ce_asc(x, idx, j, asc, flip=None):
    down = jnp.roll(x, -j, axis=0)
    up = jnp.roll(x, j, axis=0)
    lower = (idx & j) == 0
    partner = jnp.where(lower, down, up)
    take_min = asc == lower
    if flip is not None:
        take_min = jnp.logical_xor(take_min, flip)
    return jnp.where(take_min, jnp.minimum(x, partner), jnp.maximum(x, partner))


def kernel(x):
    m_per, n = x.shape
    m_full = N_DEV * m_per

    def body(x_ref, out_ref, chunk_ref, full_ref, send_sems, recv_sems):
        my_pos = lax.axis_index("i")

        barrier_sem = pltpu.get_barrier_semaphore()
        for d in (2, 1, 3):
            pl.semaphore_signal(
                barrier_sem, inc=1,
                device_id=((my_pos + d) % N_DEV,),
                device_id_type=pl.DeviceIdType.MESH,
            )

        xv = x_ref[:, :].astype(jnp.bfloat16)
        m_h = m_per // 2
        flip = (my_pos % 2) == 1
        xvf = jnp.concatenate([xv[:m_h, :], xv[m_h:, :]], axis=1)
        idx_l = lax.broadcasted_iota(jnp.int32, (m_h, 2 * n), 0)
        asc_ll = lax.broadcasted_iota(jnp.int32, (m_h, 2 * n), 1) < n
        k = 2
        while k <= m_h:
            j = k // 2
            while j >= 1:
                asc = asc_ll if k == m_h else (idx_l & k) == 0
                xvf = _ce_asc(xvf, idx_l, j, asc, flip)
                j //= 2
            k *= 2
        a = xvf[:, :n]
        b = xvf[:, n:]
        blo = jnp.where(flip, jnp.maximum(a, b), jnp.minimum(a, b))
        bhi = jnp.where(flip, jnp.minimum(a, b), jnp.maximum(a, b))
        xvf = jnp.concatenate([blo, bhi], axis=1)
        j = m_h // 2
        while j >= 1:
            xvf = _ce_asc(xvf, idx_l, j, True, flip)
            j //= 2
        xv = jnp.concatenate([xvf[:, :n], xvf[:, n:]], axis=0)
        chunk_ref[:, :] = xv
        full_ref[pl.ds(my_pos * m_per, m_per), :] = xv

        pl.semaphore_wait(barrier_sem, N_DEV - 1)

        sends = []
        for d in (2, 1, 3):
            rdma = pltpu.make_async_remote_copy(
                src_ref=chunk_ref,
                dst_ref=full_ref.at[pl.ds(my_pos * m_per, m_per)],
                send_sem=send_sems.at[d - 1],
                recv_sem=recv_sems.at[d - 1],
                device_id=((my_pos + d) % N_DEV,),
                device_id_type=pl.DeviceIdType.MESH,
            )
            rdma.start()
            sends.append(rdma)

        for d in range(1, N_DEV):
            origin = (my_pos - d) % N_DEV
            recv = pltpu.make_async_remote_copy(
                src_ref=chunk_ref,
                dst_ref=full_ref.at[pl.ds(origin * m_per, m_per)],
                send_sem=send_sems.at[d - 1],
                recv_sem=recv_sems.at[d - 1],
                device_id=(my_pos,),
                device_id_type=pl.DeviceIdType.MESH,
            )
            recv.wait_recv()

        m_half = 2 * m_per
        xf = full_ref[:, :]
        xff = jnp.concatenate([xf[:m_half, :], xf[m_half:, :]], axis=1)
        idx_hf = lax.broadcasted_iota(jnp.int32, (m_half, 2 * n), 0)
        asc_lane = lax.broadcasted_iota(jnp.int32, (m_half, 2 * n), 1) < n
        j = 128
        while j >= 1:
            xff = _ce_asc(xff, idx_hf, j, asc_lane)
            j //= 2

        lo = xff[:, :n]
        hi = xff[:, n:]
        is_lo_half = my_pos < 2
        xh = jnp.where(is_lo_half, jnp.minimum(lo, hi), jnp.maximum(lo, hi))

        a = xh[:m_per, :]
        b = xh[m_per:, :]
        qlo = jnp.minimum(a, b)
        qhi = jnp.maximum(a, b)
        xqf = jnp.concatenate([qlo, qhi], axis=1)

        idx_q = lax.broadcasted_iota(jnp.int32, (m_per, 2 * n), 0)
        j = 64
        while j >= 1:
            xqf = _ce_asc(xqf, idx_q, j, True)
            j //= 2

        is_lo_q = (my_pos % 2) == 0
        out_ref[:, :] = jnp.where(is_lo_q, xqf[:, :n], xqf[:, n:])

        for rdma in sends:
            rdma.wait_send()

    return pl.pallas_call(
        body,
        out_shape=jax.ShapeDtypeStruct((m_per, n), jnp.bfloat16),
        in_specs=[pl.BlockSpec(memory_space=pltpu.VMEM)],
        out_specs=pl.BlockSpec(memory_space=pltpu.VMEM),
        scratch_shapes=[
            pltpu.VMEM((m_per, n), jnp.bfloat16),
            pltpu.VMEM((m_full, n), jnp.bfloat16),
            pltpu.SemaphoreType.DMA((N_DEV - 1,)),
            pltpu.SemaphoreType.DMA((N_DEV - 1,)),
        ],
        compiler_params=pltpu.CompilerParams(collective_id=0),
    )(x)
